# Initial kernel scaffold; baseline (speedup 1.0000x reference)
import jax
import jax.numpy as jnp
from jax.experimental import pallas as pl


def kernel(idx, table):
    raise NotImplementedError("write your pallas kernel here")


# speedup vs baseline: 1.8383x; 1.8383x over previous
import functools

import jax
import jax.numpy as jnp
from jax import lax
from jax.experimental import pallas as pl
from jax.experimental.pallas import tpu as pltpu
from jax.experimental.pallas import tpu_sc as plsc

VOCAB = 1000000
N_EMB = 64
BATCH = 16384
HIST = 50

_B_FLAT = BATCH * HIST
_CHUNK = 128
_NW = 32
_CHUNKS_PER_W = _B_FLAT // (_CHUNK * _NW)


def _make_gather():
    mesh = plsc.VectorSubcoreMesh(core_axis_name="c", subcore_axis_name="s")

    @functools.partial(
        pl.kernel,
        mesh=mesh,
        out_type=jax.ShapeDtypeStruct((_B_FLAT // _CHUNK, _CHUNK, N_EMB),
                                      jnp.float32),
        scratch_types=[
            pltpu.VMEM((_CHUNKS_PER_W, _CHUNK), jnp.int32),
            pltpu.VMEM((2, _CHUNK, N_EMB), jnp.float32),
            pltpu.SemaphoreType.DMA,
            pltpu.SemaphoreType.DMA,
        ],
        compiler_params=pltpu.CompilerParams(use_tc_tiling_on_sc=False),
    )
    def gather_kernel(table_hbm, idx_hbm, out_hbm, idx_v, rows_v, gsem, osem):
        wid = lax.axis_index("s") * 2 + lax.axis_index("c")
        chunk0 = wid * _CHUNKS_PER_W

        pltpu.sync_copy(idx_hbm.at[pl.ds(chunk0, _CHUNKS_PER_W)], idx_v)

        pltpu.async_copy(table_hbm.at[idx_v.at[0]], rows_v.at[0], gsem)

        def body(j, _):
            buf = lax.rem(j, 2)
            nbuf = lax.rem(j + 1, 2)

            @pl.when(j >= 1)
            def _():
                pltpu.make_async_copy(rows_v.at[nbuf],
                                      out_hbm.at[chunk0 + j - 1], osem).wait()

            @pl.when(j + 1 < _CHUNKS_PER_W)
            def _():
                pltpu.async_copy(table_hbm.at[idx_v.at[j + 1]],
                                 rows_v.at[nbuf], gsem)

            pltpu.make_async_copy(table_hbm.at[idx_v.at[j]],
                                  rows_v.at[buf], gsem).wait()
            pltpu.async_copy(rows_v.at[buf], out_hbm.at[chunk0 + j], osem)

            return ()

        lax.fori_loop(0, _CHUNKS_PER_W, body, (), unroll=False)
        pltpu.make_async_copy(rows_v.at[lax.rem(_CHUNKS_PER_W - 1, 2)],
                              out_hbm.at[chunk0 + _CHUNKS_PER_W - 1],
                              osem).wait()

    return gather_kernel


_gather = _make_gather()


def kernel(idx, table):
    idx_flat = idx.reshape(_B_FLAT // _CHUNK, _CHUNK).astype(jnp.int32)
    out = _gather(table, idx_flat)
    return out.reshape(BATCH, HIST, N_EMB)


# speedup vs baseline: 1.8788x; 1.0221x over previous
import functools

import jax
import jax.numpy as jnp
from jax import lax
from jax.experimental import pallas as pl
from jax.experimental.pallas import tpu as pltpu
from jax.experimental.pallas import tpu_sc as plsc

VOCAB = 1000000
N_EMB = 64
BATCH = 16384
HIST = 50

_B_FLAT = BATCH * HIST
_CHUNK = 128
_NW = 32
_CHUNKS_PER_W = _B_FLAT // (_CHUNK * _NW)


_NBUF = 8
_DEPTH = 4
_GROUPS = _CHUNKS_PER_W // _NBUF


def _make_gather():
    mesh = plsc.VectorSubcoreMesh(core_axis_name="c", subcore_axis_name="s")

    @functools.partial(
        pl.kernel,
        mesh=mesh,
        out_type=jax.ShapeDtypeStruct((_B_FLAT // _CHUNK, _CHUNK, N_EMB),
                                      jnp.float32),
        scratch_types=[
            pltpu.VMEM((_CHUNKS_PER_W, _CHUNK), jnp.int32),
            pltpu.VMEM((_NBUF, _CHUNK, N_EMB), jnp.float32),
            pltpu.SemaphoreType.DMA,
            pltpu.SemaphoreType.DMA,
        ],
        compiler_params=pltpu.CompilerParams(use_tc_tiling_on_sc=False),
    )
    def gather_kernel(table_hbm, idx_hbm, out_hbm, idx_v, rows_v, gsem, osem):
        wid = lax.axis_index("s") * 2 + lax.axis_index("c")
        chunk0 = wid * _CHUNKS_PER_W

        pltpu.sync_copy(idx_hbm.at[pl.ds(chunk0, _CHUNKS_PER_W)], idx_v)

        for b in range(_DEPTH):
            pltpu.async_copy(table_hbm.at[idx_v.at[b]], rows_v.at[b], gsem)

        def group(g, _):
            for b in range(_NBUF):
                j = g * _NBUF + b
                gbuf = (b + _DEPTH) % _NBUF

                @pl.when(j + _DEPTH - _NBUF >= 0)
                def _():
                    pltpu.make_async_copy(
                        rows_v.at[gbuf],
                        out_hbm.at[chunk0 + j + _DEPTH - _NBUF], osem).wait()

                @pl.when(j + _DEPTH < _CHUNKS_PER_W)
                def _():
                    pltpu.async_copy(table_hbm.at[idx_v.at[j + _DEPTH]],
                                     rows_v.at[gbuf], gsem)

                pltpu.make_async_copy(table_hbm.at[idx_v.at[j]],
                                      rows_v.at[b], gsem).wait()
                pltpu.async_copy(rows_v.at[b], out_hbm.at[chunk0 + j], osem)
            return ()

        lax.fori_loop(0, _GROUPS, group, (), unroll=False)

        for t in range(_CHUNKS_PER_W + _DEPTH - _NBUF, _CHUNKS_PER_W):
            pltpu.make_async_copy(rows_v.at[t % _NBUF],
                                  out_hbm.at[chunk0 + t], osem).wait()

    return gather_kernel


_gather = _make_gather()


def kernel(idx, table):
    idx_flat = idx.reshape(_B_FLAT // _CHUNK, _CHUNK).astype(jnp.int32)
    out = _gather(table, idx_flat)
    return out.reshape(BATCH, HIST, N_EMB)
